# Initial kernel scaffold; baseline (speedup 1.0000x reference)
import jax
import jax.numpy as jnp
from jax.experimental import pallas as pl


def kernel(x, order):
    raise NotImplementedError("write your pallas kernel here")


# speedup vs baseline: 1.7924x; 1.7924x over previous
import functools

import jax
import jax.numpy as jnp
from jax import lax
from jax.experimental import pallas as pl
from jax.experimental.pallas import tpu as pltpu
from jax.experimental.pallas import tpu_sc as plsc

_ROWS = 16384
_COLS = 128

_info = plsc.get_sparse_core_info()
_NC, _NS, _L = _info.num_cores, _info.num_subcores, _info.num_lanes
_NW = _NC * _NS
_RW = _ROWS // _NW
_WSZ = _RW * _COLS
_NGRP = _COLS // _L

_mesh = plsc.VectorSubcoreMesh(core_axis_name="c", subcore_axis_name="s")


@functools.partial(
    pl.kernel,
    mesh=_mesh,
    out_type=jax.ShapeDtypeStruct((_ROWS * _COLS,), jnp.float32),
    scratch_types=[
        pltpu.VMEM((_WSZ,), jnp.float32),
        pltpu.VMEM((_COLS,), jnp.int32),
    ],
    compiler_params=pltpu.CompilerParams(needs_layout_passes=False),
)
def _rearrange(x_hbm, order_hbm, out_hbm, buf, idx_v):
    wid = lax.axis_index("s") * _NC + lax.axis_index("c")
    base = wid * _WSZ

    pltpu.sync_copy(order_hbm, idx_v)
    pltpu.sync_copy(x_hbm.at[pl.ds(base, _WSZ)], buf)

    gidx = [idx_v[pl.ds(_L * k, _L)] for k in range(_NGRP)]

    def row_body(r, carry):
        off = r * _COLS
        vals = [plsc.load_gather(buf, [off + gidx[k]]) for k in range(_NGRP)]
        for k in range(_NGRP):
            buf[pl.ds(off + _L * k, _L)] = vals[k]
        return carry

    lax.fori_loop(0, _RW, row_body, 0)

    pltpu.sync_copy(buf, out_hbm.at[pl.ds(base, _WSZ)])


def kernel(x, order):
    out_flat = _rearrange(x.reshape(-1), order)
    return out_flat.reshape(_ROWS, _COLS)


# speedup vs baseline: 1.8806x; 1.0492x over previous
import functools

import jax
import jax.numpy as jnp
from jax import lax
from jax.experimental import pallas as pl
from jax.experimental.pallas import tpu as pltpu
from jax.experimental.pallas import tpu_sc as plsc

_ROWS = 16384
_COLS = 128

_info = plsc.get_sparse_core_info()
_NC, _NS, _L = _info.num_cores, _info.num_subcores, _info.num_lanes
_NW = _NC * _NS
_RW = _ROWS // _NW
_WSZ = _RW * _COLS
_NGRP = _COLS // _L

_CH = 64
_CW = _CH * _COLS
_NCH = _RW // _CH

_mesh = plsc.VectorSubcoreMesh(core_axis_name="c", subcore_axis_name="s")


@functools.partial(
    pl.kernel,
    mesh=_mesh,
    out_type=jax.ShapeDtypeStruct((_ROWS * _COLS,), jnp.float32),
    scratch_types=[
        pltpu.VMEM((_CW,), jnp.float32),
        pltpu.VMEM((_CW,), jnp.float32),
        pltpu.VMEM((_CW,), jnp.float32),
        pltpu.VMEM((_CW,), jnp.float32),
        pltpu.VMEM((_COLS,), jnp.int32),
        pltpu.SemaphoreType.DMA,
        pltpu.SemaphoreType.DMA,
        pltpu.SemaphoreType.DMA,
        pltpu.SemaphoreType.DMA,
    ],
    compiler_params=pltpu.CompilerParams(needs_layout_passes=False),
)
def _rearrange(x_hbm, order_hbm, out_hbm, in0, in1, ou0, ou1, idx_v,
               si0, si1, so0, so1):
    wid = lax.axis_index("s") * _NC + lax.axis_index("c")
    base = wid * _WSZ

    pltpu.sync_copy(order_hbm, idx_v)
    gidx = [idx_v[pl.ds(_L * k, _L)] for k in range(_NGRP)]

    ins, outs = [in0, in1], [ou0, ou1]
    isem, osem = [si0, si1], [so0, so1]

    def in_copy(c):
        return pltpu.make_async_copy(
            x_hbm.at[pl.ds(base + c * _CW, _CW)], ins[c % 2], isem[c % 2])

    def out_copy(c):
        return pltpu.make_async_copy(
            outs[c % 2], out_hbm.at[pl.ds(base + c * _CW, _CW)], osem[c % 2])

    in_copy(0).start()
    in_copy(1).start()
    for c in range(_NCH):
        in_copy(c).wait()
        if c >= 2:
            out_copy(c - 2).wait()
        src, dst = ins[c % 2], outs[c % 2]

        @plsc.parallel_loop(0, _CH, unroll=4)
        def _row(r):
            off = r * _COLS
            vals = [plsc.load_gather(src, [off + gidx[k]])
                    for k in range(_NGRP)]
            for k in range(_NGRP):
                dst[pl.ds(off + _L * k, _L)] = vals[k]

        out_copy(c).start()
        if c + 2 < _NCH:
            in_copy(c + 2).start()

    out_copy(_NCH - 2).wait()
    out_copy(_NCH - 1).wait()


def kernel(x, order):
    out_flat = _rearrange(x.reshape(-1), order)
    return out_flat.reshape(_ROWS, _COLS)
